# baseline (speedup 1.0000x reference)
import functools

import jax
import jax.numpy as jnp
from jax import lax
from jax.experimental import pallas as pl
from jax.experimental.pallas import tpu as pltpu
from jax.experimental.pallas import tpu_sc as plsc

B = 16384
D = 64
NW = 32
BPW = B // NW
CHUNK = 128
NCH = BPW // CHUNK
HALF = 128
NH = BPW // HALF
L = 16


def _fold_body(w1, w2, w3, wp, b1, b2, b3, bp, w_out, c_out):
    wpa = wp[0:32, :]
    v3 = jnp.dot(w3[...], wpa, preferred_element_type=jnp.float32)
    v2 = jnp.dot(w2[...], v3, preferred_element_type=jnp.float32)
    w = jnp.dot(w1[...], v2, preferred_element_type=jnp.float32)
    c = (jnp.sum(b1[...] * v2) + jnp.sum(b2[...] * v3)
         + jnp.sum(b3[...] * wpa) + bp[0, 0])
    w_out[...] = w
    c_out[...] = jnp.broadcast_to(c, (1, 1))


def _fold(W1, b1, W2, b2, W3, b3, Wp, bp):
    return pl.pallas_call(
        _fold_body,
        out_shape=(
            jax.ShapeDtypeStruct((128, 1), jnp.float32),
            jax.ShapeDtypeStruct((1, 1), jnp.float32),
        ),
    )(W1, W2, W3, Wp, b1.reshape(128, 1), b2.reshape(64, 1),
      b3.reshape(32, 1), bp.reshape(1, 1))


def _worker_id():
    return lax.axis_index("s") * 2 + lax.axis_index("c")


def _item_body(item, tab_mi, tab_fi, wpack, part, fi_out,
               idxi, bmi, bfi, wv, partbuf, sem):
    base = _worker_id() * BPW

    pltpu.sync_copy(wpack, wv)
    wi = [wv[0, pl.ds(64 + k * L, L)] for k in range(4)]

    for ch in range(NCH):
        pltpu.sync_copy(item.at[pl.ds(base + ch * CHUNK, CHUNK)],
                        idxi.at[ch])

    rowids = lax.iota(jnp.int32, L)
    perms = {sh: rowids ^ sh for sh in (8, 4, 2, 1)}

    def start(ch):
        par = ch % 2
        return [
            pltpu.async_copy(tab_mi.at[idxi.at[ch]], bmi[par], sem),
            pltpu.async_copy(tab_fi.at[idxi.at[ch]], bfi[par], sem),
        ]

    handles = start(0)
    for ch in range(NCH):
        nxt = start(ch + 1) if ch + 1 < NCH else None
        for h in handles:
            h.wait()
        par = ch % 2
        mi, fi = bmi[par], bfi[par]

        def block(g, carry, mi=mi, ch=ch):
            s = rowids * 0.0
            for r in range(L):
                row = g * L + r
                acc = mi[row, pl.ds(0, L)] * wi[0]
                for k in range(1, 4):
                    acc = acc + mi[row, pl.ds(k * L, L)] * wi[k]
                for sh in (8, 4, 2, 1):
                    acc = acc + acc.at[perms[sh]].get(
                        mode="promise_in_bounds")
                s = jnp.where(rowids == r, acc, s)
            partbuf[pl.ds(ch * CHUNK + g * L, L)] = s
            return carry

        lax.fori_loop(0, CHUNK // L, block, 0)
        pltpu.sync_copy(fi, fi_out.at[pl.ds(base + ch * CHUNK, CHUNK), :])
        handles = nxt

    pltpu.sync_copy(partbuf, part.at[pl.ds(base, BPW)])


def _sc_items(item, tab_mi, tab_fi, wpack):
    mesh = plsc.VectorSubcoreMesh(core_axis_name="c", subcore_axis_name="s")
    scratch = (
        pltpu.VMEM((NCH, CHUNK), jnp.int32),
        [pltpu.VMEM((CHUNK, D), jnp.float32) for _ in range(2)],
        [pltpu.VMEM((CHUNK, D), jnp.float32) for _ in range(2)],
        pltpu.VMEM((2, 128), jnp.float32),
        pltpu.VMEM((BPW,), jnp.float32),
        pltpu.SemaphoreType.DMA,
    )
    f = pl.kernel(
        _item_body,
        out_type=(
            jax.ShapeDtypeStruct((B,), jnp.float32),
            jax.ShapeDtypeStruct((B, D), jnp.float32),
        ),
        mesh=mesh,
        scratch_types=scratch,
        compiler_params=pltpu.CompilerParams(use_tc_tiling_on_sc=False),
    )
    return f(item, tab_mi, tab_fi, wpack)


def _user_body(user, tab_mu, tab_fu, part, fi, wpack, out,
               idxu, rmu, rfu, bfi, partbuf, wv, outbuf, sem0, sem1):
    base = _worker_id() * BPW

    pltpu.sync_copy(wpack, wv)
    wu = [wv[0, pl.ds(k * L, L)] for k in range(4)]
    wb = [wv[1, pl.ds(k * L, L)] for k in range(4)]
    cvec = wv[1, pl.ds(64, L)]

    pltpu.sync_copy(user.at[pl.ds(base, BPW)], idxu)
    pltpu.sync_copy(part.at[pl.ds(base, BPW)], partbuf)

    rowids = lax.iota(jnp.int32, L)
    perms = {sh: rowids ^ sh for sh in (8, 4, 2, 1)}

    def fire(g, carry, h=None):
        uvals = idxu[pl.ds(h * HALF + g * L, L)]
        for r in range(L):
            u = uvals[r]
            dst = pl.ds(g * L + r, 1)
            pltpu.async_copy(tab_mu.at[pl.ds(u, 1), :], rmu.at[dst, :], sem0)
            pltpu.async_copy(tab_fu.at[pl.ds(u, 1), :], rfu.at[dst, :], sem1)
        return carry

    def compute(g, carry, h=None):
        s = partbuf[pl.ds(h * HALF + g * L, L)] + cvec
        for r in range(L):
            row = g * L + r
            acc = rmu[row, pl.ds(0, L)] * wu[0]
            for k in range(1, 4):
                acc = acc + rmu[row, pl.ds(k * L, L)] * wu[k]
            for k in range(4):
                acc = acc + (rfu[row, pl.ds(k * L, L)]
                             * bfi[row, pl.ds(k * L, L)]) * wb[k]
            for sh in (8, 4, 2, 1):
                acc = acc + acc.at[perms[sh]].get(mode="promise_in_bounds")
            s = s + jnp.where(rowids == r, acc, 0.0)
        o = 1.0 / (1.0 + jnp.exp(-s))
        outbuf[pl.ds(h * HALF + g * L, L)] = o
        return carry

    for h in range(NH):
        pltpu.sync_copy(fi.at[pl.ds(base + h * HALF, HALF), :], bfi)
        lax.fori_loop(0, HALF // L, functools.partial(fire, h=h), 0)
        pltpu.make_async_copy(
            tab_mu.at[pl.ds(0, HALF), :], rmu, sem0).wait()
        pltpu.make_async_copy(
            tab_fu.at[pl.ds(0, HALF), :], rfu, sem1).wait()
        lax.fori_loop(0, HALF // L, functools.partial(compute, h=h), 0)

    pltpu.sync_copy(outbuf, out.at[pl.ds(base, BPW)])


def _sc_users(user, tab_mu, tab_fu, part, fi, wpack):
    mesh = plsc.VectorSubcoreMesh(core_axis_name="c", subcore_axis_name="s")
    scratch = (
        pltpu.VMEM((BPW,), jnp.int32),
        pltpu.VMEM((HALF, D), jnp.float32),
        pltpu.VMEM((HALF, D), jnp.float32),
        pltpu.VMEM((HALF, D), jnp.float32),
        pltpu.VMEM((BPW,), jnp.float32),
        pltpu.VMEM((2, 128), jnp.float32),
        pltpu.VMEM((BPW,), jnp.float32),
        pltpu.SemaphoreType.DMA,
        pltpu.SemaphoreType.DMA,
    )
    f = pl.kernel(
        _user_body,
        out_type=jax.ShapeDtypeStruct((B,), jnp.float32),
        mesh=mesh,
        scratch_types=scratch,
    )
    return f(user, tab_mu, tab_fu, part, fi, wpack)


def kernel(user, item, mlp_user_table, mf_user_table, mlp_item_table,
           mf_item_table, W1, b1, W2, b2, W3, b3, Wp, bp):
    w2d, c2d = _fold(W1, b1, W2, b2, W3, b3, Wp, bp)
    row1 = jnp.concatenate(
        [Wp[32:, 0], jnp.broadcast_to(c2d[0, 0], (64,))])
    wpack = jnp.stack([w2d[:, 0], row1])
    part, fi = _sc_items(item.astype(jnp.int32), mlp_item_table,
                         mf_item_table, wpack)
    out = _sc_users(user.astype(jnp.int32), mlp_user_table, mf_user_table,
                    part, fi, wpack)
    return out.reshape(B, 1)


# speedup vs baseline: 1.0008x; 1.0008x over previous
import functools

import jax
import jax.numpy as jnp
from jax import lax
from jax.experimental import pallas as pl
from jax.experimental.pallas import tpu as pltpu
from jax.experimental.pallas import tpu_sc as plsc

B = 16384
D = 64
NW = 32
BPW = B // NW
CHUNK = 128
NCH = BPW // CHUNK
HALF = 128
NH = BPW // HALF
L = 16


def _fold_body(w1, w2, w3, wp, b1, b2, b3, bp, w_out, c_out):
    wpa = wp[0:32, :]
    v3 = jnp.dot(w3[...], wpa, preferred_element_type=jnp.float32)
    v2 = jnp.dot(w2[...], v3, preferred_element_type=jnp.float32)
    w = jnp.dot(w1[...], v2, preferred_element_type=jnp.float32)
    c = (jnp.sum(b1[...] * v2) + jnp.sum(b2[...] * v3)
         + jnp.sum(b3[...] * wpa) + bp[0, 0])
    w_out[...] = w
    c_out[...] = jnp.broadcast_to(c, (1, 1))


def _fold(W1, b1, W2, b2, W3, b3, Wp, bp):
    return pl.pallas_call(
        _fold_body,
        out_shape=(
            jax.ShapeDtypeStruct((128, 1), jnp.float32),
            jax.ShapeDtypeStruct((1, 1), jnp.float32),
        ),
    )(W1, W2, W3, Wp, b1.reshape(128, 1), b2.reshape(64, 1),
      b3.reshape(32, 1), bp.reshape(1, 1))


def _worker_id():
    return lax.axis_index("s") * 2 + lax.axis_index("c")


def _item_body(item, tab_mi, tab_fi, wpack, part, fi_out,
               idxi, bmi, bfi, wv, partbuf, sem):
    base = _worker_id() * BPW

    pltpu.sync_copy(wpack, wv)
    wi = [wv[0, pl.ds(64 + k * L, L)] for k in range(4)]

    for ch in range(NCH):
        pltpu.sync_copy(item.at[pl.ds(base + ch * CHUNK, CHUNK)],
                        idxi.at[ch])

    rowids = lax.iota(jnp.int32, L)
    perms = {sh: rowids ^ sh for sh in (8, 4, 2, 1)}

    def start(ch):
        par = ch % 2
        return [
            pltpu.async_copy(tab_mi.at[idxi.at[ch]], bmi[par], sem),
            pltpu.async_copy(tab_fi.at[idxi.at[ch]], bfi[par], sem),
        ]

    handles = start(0)
    for ch in range(NCH):
        nxt = start(ch + 1) if ch + 1 < NCH else None
        for h in handles:
            h.wait()
        par = ch % 2
        mi, fi = bmi[par], bfi[par]

        def block(g, carry, mi=mi, ch=ch):
            s = rowids * 0.0
            for r in range(L):
                row = g * L + r
                acc = mi[row, pl.ds(0, L)] * wi[0]
                for k in range(1, 4):
                    acc = acc + mi[row, pl.ds(k * L, L)] * wi[k]
                for sh in (8, 4, 2, 1):
                    acc = acc + acc.at[perms[sh]].get(
                        mode="promise_in_bounds")
                s = jnp.where(rowids == r, acc, s)
            partbuf[pl.ds(ch * CHUNK + g * L, L)] = s
            return carry

        lax.fori_loop(0, CHUNK // L, block, 0)
        pltpu.sync_copy(fi, fi_out.at[pl.ds(base + ch * CHUNK, CHUNK), :])
        handles = nxt

    pltpu.sync_copy(partbuf, part.at[pl.ds(base, BPW)])


def _sc_items(item, tab_mi, tab_fi, wpack):
    mesh = plsc.VectorSubcoreMesh(core_axis_name="c", subcore_axis_name="s")
    scratch = (
        pltpu.VMEM((NCH, CHUNK), jnp.int32),
        [pltpu.VMEM((CHUNK, D), jnp.float32) for _ in range(2)],
        [pltpu.VMEM((CHUNK, D), jnp.float32) for _ in range(2)],
        pltpu.VMEM((2, 128), jnp.float32),
        pltpu.VMEM((BPW,), jnp.float32),
        pltpu.SemaphoreType.DMA,
    )
    f = pl.kernel(
        _item_body,
        out_type=(
            jax.ShapeDtypeStruct((B,), jnp.float32),
            jax.ShapeDtypeStruct((B, D), jnp.float32),
        ),
        mesh=mesh,
        scratch_types=scratch,
        compiler_params=pltpu.CompilerParams(
            use_tc_tiling_on_sc=False, skip_device_barrier=True),
    )
    return f(item, tab_mi, tab_fi, wpack)


def _user_body(user, tab_mu, tab_fu, part, fi, wpack, out,
               idxu, rmu, rfu, bfi, partbuf, wv, outbuf, sem0, sem1):
    base = _worker_id() * BPW

    pltpu.sync_copy(wpack, wv)
    wu = [wv[0, pl.ds(k * L, L)] for k in range(4)]
    wb = [wv[1, pl.ds(k * L, L)] for k in range(4)]
    cvec = wv[1, pl.ds(64, L)]

    pltpu.sync_copy(user.at[pl.ds(base, BPW)], idxu)
    pltpu.sync_copy(part.at[pl.ds(base, BPW)], partbuf)

    rowids = lax.iota(jnp.int32, L)
    perms = {sh: rowids ^ sh for sh in (8, 4, 2, 1)}

    def fire(g, carry, h=None):
        uvals = idxu[pl.ds(h * HALF + g * L, L)]
        for r in range(L):
            u = uvals[r]
            dst = pl.ds(g * L + r, 1)
            pltpu.async_copy(tab_mu.at[pl.ds(u, 1), :], rmu.at[dst, :], sem0)
            pltpu.async_copy(tab_fu.at[pl.ds(u, 1), :], rfu.at[dst, :], sem1)
        return carry

    def compute(g, carry, h=None):
        s = partbuf[pl.ds(h * HALF + g * L, L)] + cvec
        for r in range(L):
            row = g * L + r
            acc = rmu[row, pl.ds(0, L)] * wu[0]
            for k in range(1, 4):
                acc = acc + rmu[row, pl.ds(k * L, L)] * wu[k]
            for k in range(4):
                acc = acc + (rfu[row, pl.ds(k * L, L)]
                             * bfi[row, pl.ds(k * L, L)]) * wb[k]
            for sh in (8, 4, 2, 1):
                acc = acc + acc.at[perms[sh]].get(mode="promise_in_bounds")
            s = s + jnp.where(rowids == r, acc, 0.0)
        o = 1.0 / (1.0 + jnp.exp(-s))
        outbuf[pl.ds(h * HALF + g * L, L)] = o
        return carry

    for h in range(NH):
        pltpu.sync_copy(fi.at[pl.ds(base + h * HALF, HALF), :], bfi)
        lax.fori_loop(0, HALF // L, functools.partial(fire, h=h), 0)
        pltpu.make_async_copy(
            tab_mu.at[pl.ds(0, HALF), :], rmu, sem0).wait()
        pltpu.make_async_copy(
            tab_fu.at[pl.ds(0, HALF), :], rfu, sem1).wait()
        lax.fori_loop(0, HALF // L, functools.partial(compute, h=h), 0)

    pltpu.sync_copy(outbuf, out.at[pl.ds(base, BPW)])


def _sc_users(user, tab_mu, tab_fu, part, fi, wpack):
    mesh = plsc.VectorSubcoreMesh(core_axis_name="c", subcore_axis_name="s")
    scratch = (
        pltpu.VMEM((BPW,), jnp.int32),
        pltpu.VMEM((HALF, D), jnp.float32),
        pltpu.VMEM((HALF, D), jnp.float32),
        pltpu.VMEM((HALF, D), jnp.float32),
        pltpu.VMEM((BPW,), jnp.float32),
        pltpu.VMEM((2, 128), jnp.float32),
        pltpu.VMEM((BPW,), jnp.float32),
        pltpu.SemaphoreType.DMA,
        pltpu.SemaphoreType.DMA,
    )
    f = pl.kernel(
        _user_body,
        out_type=jax.ShapeDtypeStruct((B,), jnp.float32),
        mesh=mesh,
        scratch_types=scratch,
        compiler_params=pltpu.CompilerParams(skip_device_barrier=True),
    )
    return f(user, tab_mu, tab_fu, part, fi, wpack)


def kernel(user, item, mlp_user_table, mf_user_table, mlp_item_table,
           mf_item_table, W1, b1, W2, b2, W3, b3, Wp, bp):
    w2d, c2d = _fold(W1, b1, W2, b2, W3, b3, Wp, bp)
    row1 = jnp.concatenate(
        [Wp[32:, 0], jnp.broadcast_to(c2d[0, 0], (64,))])
    wpack = jnp.stack([w2d[:, 0], row1])
    part, fi = _sc_items(item.astype(jnp.int32), mlp_item_table,
                         mf_item_table, wpack)
    out = _sc_users(user.astype(jnp.int32), mlp_user_table, mf_user_table,
                    part, fi, wpack)
    return out.reshape(B, 1)


# speedup vs baseline: 1.0422x; 1.0414x over previous
import functools

import jax
import jax.numpy as jnp
from jax import lax
from jax.experimental import pallas as pl
from jax.experimental.pallas import tpu as pltpu
from jax.experimental.pallas import tpu_sc as plsc

B = 16384
D = 64
NW = 32
BPW = B // NW
HALF = 128
NH = BPW // HALF
L = 16


def _fold_body(w1, w2, w3, wp, b1, b2, b3, bp, w_out, c_out):
    wpa = wp[0:32, :]
    v3 = jnp.dot(w3[...], wpa, preferred_element_type=jnp.float32)
    v2 = jnp.dot(w2[...], v3, preferred_element_type=jnp.float32)
    w = jnp.dot(w1[...], v2, preferred_element_type=jnp.float32)
    c = (jnp.sum(b1[...] * v2) + jnp.sum(b2[...] * v3)
         + jnp.sum(b3[...] * wpa) + bp[0, 0])
    w_out[...] = w
    c_out[...] = jnp.broadcast_to(c, (1, 1))


def _fold(W1, b1, W2, b2, W3, b3, Wp, bp):
    return pl.pallas_call(
        _fold_body,
        out_shape=(
            jax.ShapeDtypeStruct((128, 1), jnp.float32),
            jax.ShapeDtypeStruct((1, 1), jnp.float32),
        ),
    )(W1, W2, W3, Wp, b1.reshape(128, 1), b2.reshape(64, 1),
      b3.reshape(32, 1), bp.reshape(1, 1))


def _sc_body(user, item, tab_mu, tab_fu, tab_mi, tab_fi, wpack, out,
             idxu, idxi, rmu, rmi, rfu, rfi, wv, outbuf,
             sem0, sem1, sem2, sem3):
    wid = lax.axis_index("s") * 2 + lax.axis_index("c")
    base = wid * BPW

    pltpu.sync_copy(wpack, wv)
    wu = [wv[0, pl.ds(k * L, L)] for k in range(4)]
    wi = [wv[0, pl.ds(64 + k * L, L)] for k in range(4)]
    wb = [wv[1, pl.ds(k * L, L)] for k in range(4)]
    cvec = wv[1, pl.ds(64, L)]

    pltpu.sync_copy(user.at[pl.ds(base, BPW)], idxu)
    pltpu.sync_copy(item.at[pl.ds(base, BPW)], idxi)

    rowids = lax.iota(jnp.int32, L)
    perms = {sh: rowids ^ sh for sh in (8, 4, 2, 1)}

    def fire(g, carry, h=None):
        uvals = idxu[pl.ds(h * HALF + g * L, L)]
        ivals = idxi[pl.ds(h * HALF + g * L, L)]
        for r in range(L):
            u = uvals[r]
            it = ivals[r]
            dst = pl.ds(g * L + r, 1)
            pltpu.async_copy(tab_mu.at[pl.ds(u, 1), :], rmu.at[dst, :], sem0)
            pltpu.async_copy(tab_fu.at[pl.ds(u, 1), :], rfu.at[dst, :], sem1)
            pltpu.async_copy(tab_mi.at[pl.ds(it, 1), :], rmi.at[dst, :], sem2)
            pltpu.async_copy(tab_fi.at[pl.ds(it, 1), :], rfi.at[dst, :], sem3)
        return carry

    def compute(g, carry, h=None):
        s = cvec
        for r in range(L):
            row = g * L + r
            acc = rmu[row, pl.ds(0, L)] * wu[0]
            for k in range(1, 4):
                acc = acc + rmu[row, pl.ds(k * L, L)] * wu[k]
            for k in range(4):
                acc = acc + rmi[row, pl.ds(k * L, L)] * wi[k]
            for k in range(4):
                acc = acc + (rfu[row, pl.ds(k * L, L)]
                             * rfi[row, pl.ds(k * L, L)]) * wb[k]
            for sh in (8, 4, 2, 1):
                acc = acc + acc.at[perms[sh]].get(mode="promise_in_bounds")
            s = jnp.where(rowids == r, acc, s)
        o = 1.0 / (1.0 + jnp.exp(-s))
        outbuf[pl.ds(h * HALF + g * L, L)] = o
        return carry

    for h in range(NH):
        lax.fori_loop(0, HALF // L, functools.partial(fire, h=h), 0)
        pltpu.make_async_copy(tab_mu.at[pl.ds(0, HALF), :], rmu, sem0).wait()
        pltpu.make_async_copy(tab_fu.at[pl.ds(0, HALF), :], rfu, sem1).wait()
        pltpu.make_async_copy(tab_mi.at[pl.ds(0, HALF), :], rmi, sem2).wait()
        pltpu.make_async_copy(tab_fi.at[pl.ds(0, HALF), :], rfi, sem3).wait()
        lax.fori_loop(0, HALF // L, functools.partial(compute, h=h), 0)

    pltpu.sync_copy(outbuf, out.at[pl.ds(base, BPW)])


@functools.partial(jax.jit, static_argnums=())
def _sc_forward(user, item, tab_mu, tab_fu, tab_mi, tab_fi, wpack):
    mesh = plsc.VectorSubcoreMesh(core_axis_name="c", subcore_axis_name="s")
    scratch = (
        pltpu.VMEM((BPW,), jnp.int32),
        pltpu.VMEM((BPW,), jnp.int32),
        pltpu.VMEM((HALF, D), jnp.float32),
        pltpu.VMEM((HALF, D), jnp.float32),
        pltpu.VMEM((HALF, D), jnp.float32),
        pltpu.VMEM((HALF, D), jnp.float32),
        pltpu.VMEM((2, 128), jnp.float32),
        pltpu.VMEM((BPW,), jnp.float32),
        pltpu.SemaphoreType.DMA,
        pltpu.SemaphoreType.DMA,
        pltpu.SemaphoreType.DMA,
        pltpu.SemaphoreType.DMA,
    )
    f = pl.kernel(
        _sc_body,
        out_type=jax.ShapeDtypeStruct((B,), jnp.float32),
        mesh=mesh,
        scratch_types=scratch,
    )
    return f(user, item, tab_mu, tab_fu, tab_mi, tab_fi, wpack)


def kernel(user, item, mlp_user_table, mf_user_table, mlp_item_table,
           mf_item_table, W1, b1, W2, b2, W3, b3, Wp, bp):
    w2d, c2d = _fold(W1, b1, W2, b2, W3, b3, Wp, bp)
    row1 = jnp.concatenate(
        [Wp[32:, 0], jnp.broadcast_to(c2d[0, 0], (64,))])
    wpack = jnp.stack([w2d[:, 0], row1])
    out = _sc_forward(user.astype(jnp.int32), item.astype(jnp.int32),
                      mlp_user_table, mf_user_table, mlp_item_table,
                      mf_item_table, wpack)
    return out.reshape(B, 1)
